# baseline (speedup 1.0000x reference)
import functools

import jax
import jax.numpy as jnp
from jax import lax
from jax.experimental import pallas as pl
from jax.experimental.pallas import tpu as pltpu
from jax.experimental.pallas import tpu_sc as plsc

_EMB = 64
_SEQ = 200
_BATCH = 1024
_NW = 32
_CHUNK = 128
_NCHUNK = _SEQ * _BATCH // _CHUNK
_CPW = _NCHUNK // _NW
_LANES = 16
_NBUF = 4
_SHI = _SEQ // 8
_BHI = _BATCH // _CHUNK
_GRP = 7


def _pos_enc() -> jax.Array:
    i = jnp.arange(_SEQ, dtype=jnp.float32)[:, None]
    j = jnp.arange(_EMB)
    even = (j % 2 == 0)
    exponent = jnp.where(even, j, j - 1).astype(jnp.float32) / float(_EMB)
    angle = i / (10000.0 ** exponent)
    return jnp.where(even[None, :], jnp.sin(angle), jnp.cos(angle))


_MESH = plsc.VectorSubcoreMesh(core_axis_name="c", subcore_axis_name="s")

@functools.partial(
    pl.kernel,
    mesh=_MESH,
    out_type=jax.ShapeDtypeStruct((_SEQ, 8, _BHI, 8, _CHUNK), jnp.float32),
    scratch_types=[
        pltpu.VMEM((_GRP, 8, _CHUNK), jnp.int32),
        pltpu.VMEM((_GRP * 8, 1, _CHUNK), jnp.int32),
        pltpu.VMEM((_CHUNK, _CHUNK), jnp.float32),
        pltpu.VMEM((_CHUNK, _CHUNK), jnp.float32),
        pltpu.VMEM((_CHUNK, _CHUNK), jnp.float32),
        pltpu.VMEM((_CHUNK, _CHUNK), jnp.float32),
        pltpu.VMEM((8, 8, _CHUNK), jnp.float32),
        pltpu.VMEM((8, 8, _CHUNK), jnp.float32),
        pltpu.VMEM((8, 8, _CHUNK), jnp.float32),
        pltpu.VMEM((8, 8, _CHUNK), jnp.float32),
        pltpu.VMEM((_SEQ // 2, _CHUNK), jnp.float32),
    ]
    + [pltpu.SemaphoreType.DMA] * (2 * _NBUF),
    compiler_params=pltpu.CompilerParams(
        use_tc_tiling_on_sc=True, needs_layout_passes=False),
)
def _emb_kernel(x_hbm, tab_hbm, pe_hbm, out_hbm,
                idx_v, pidx_v, r0, r1, r2, r3, t0, t1, t2, t3, pe_v, *sems):
    rows = (r0, r1, r2, r3)
    tbuf = (t0, t1, t2, t3)
    gsem = sems[:_NBUF]
    osem = sems[_NBUF:]
    wid = lax.axis_index("s") * 2 + lax.axis_index("c")
    c0 = wid * _CPW
    g_a = c0 // 8
    off0 = c0 - g_a * 8
    pltpu.sync_copy(pe_hbm, pe_v)
    pltpu.sync_copy(x_hbm.at[pl.ds(g_a, _GRP)], idx_v)

    def halve(r, carry):
        for g in range(_CHUNK // _LANES):
            sl = pl.ds(g * _LANES, _LANES)
            iv = idx_v[r // 8, r % 8, sl]
            pidx_v[r, 0, sl] = iv - jnp.where(
                iv >= _VHALF, jnp.int32(_VHALF), jnp.int32(0))
        return carry

    lax.fori_loop(0, _GRP * 8, halve, 0, unroll=2)

    iota16 = lax.iota(jnp.int32, _LANES)
    bidx = [g * _LANES + iota16 for g in range(_CHUNK // _LANES)]

    def _gather(k, b):
        pltpu.async_copy(tab_hbm.at[pidx_v.at[off0 + k, 0]], rows[b], gsem[b])

    def _gather_wait(k, b):
        pltpu.make_async_copy(
            tab_hbm.at[pidx_v.at[off0 + k, 0]], rows[b], gsem[b]).wait()

    def _out_slice(k):
        c = c0 + k
        s = (c // (8 * _BHI)) * 8 + c % 8
        tc = (c // 8) % _BHI
        return s, tc

    def _out(k, b):
        s, tc = _out_slice(k)
        pltpu.async_copy(tbuf[b], out_hbm.at[s, :, tc], osem[b])

    def _out_wait(k, b):
        s, tc = _out_slice(k)
        pltpu.make_async_copy(tbuf[b], out_hbm.at[s, :, tc], osem[b]).wait()

    def _transpose_add(k, b):
        s, _ = _out_slice(k)
        ps = s // 2
        pcol = (s % 2) * _EMB
        r = off0 + k
        pard = [jnp.where(
            idx_v[r // 8, r % 8, pl.ds(g * _LANES, _LANES)] >= _VHALF,
            jnp.int32(_EMB), jnp.int32(0))
                for g in range(_CHUNK // _LANES)]

        @plsc.parallel_loop(0, _EMB, 1, unroll=2)
        def tp(d):
            pe_b = plsc.load_gather(
                pe_v, [jnp.full((_LANES,), ps, jnp.int32),
                       jnp.full((_LANES,), pcol + d, jnp.int32)])
            tr = d // 8
            dsub = d % 8
            for g in range(_CHUNK // _LANES):
                vals = plsc.load_gather(rows[b], [bidx[g], pard[g] + d])
                tbuf[b][tr, dsub, pl.ds(g * _LANES, _LANES)] = vals + pe_b

    _gather(0, 0)
    _gather(1, 1)

    def outer(kk, carry):
        for b in range(_NBUF):
            k = kk * _NBUF + b

            @pl.when(kk >= 1)
            def _():
                _out_wait(k - _NBUF, b)

            _gather(k + 2, (b + 2) % _NBUF)
            _gather_wait(k, b)
            _transpose_add(k, b)
            _out(k, b)
        return carry

    lax.fori_loop(0, (_CPW - 2) // _NBUF, outer, 0)
    for k, b in ((_CPW - 2, 0), (_CPW - 1, 1)):
        _out_wait(k - _NBUF, b)
        _gather_wait(k, b)
        _transpose_add(k, b)
        _out(k, b)
    for k, b in ((_CPW - 4, 2), (_CPW - 3, 3), (_CPW - 2, 0), (_CPW - 1, 1)):
        _out_wait(k, b)


_VHALF = 499968


def _tc_detile_body(a_ref, b_ref, out_ref):
    out_ref[...] = jnp.concatenate([a_ref[...].T, b_ref[...].T], axis=1)


def _tc_detile(tabT):
    return pl.pallas_call(
        _tc_detile_body,
        out_shape=jax.ShapeDtypeStruct((_VHALF + 2 * _CHUNK, _CHUNK),
                                       jnp.float32),
        grid=(_VHALF // _CHUNK + 1,),
        in_specs=[pl.BlockSpec((_EMB, _CHUNK), lambda i: (0, i)),
                  pl.BlockSpec((_EMB, _CHUNK),
                               lambda i: (0, i + _VHALF // _CHUNK))],
        out_specs=pl.BlockSpec((_CHUNK, _CHUNK), lambda i: (i, 0)),
    )(tabT, tabT)


def kernel(x, table):
    pe2 = _pos_enc().reshape(_SEQ // 2, _CHUNK)
    x3 = (x.reshape(_SHI, 8, _BHI, _CHUNK)
          .transpose(0, 2, 1, 3)
          .reshape(_SEQ, 8, _CHUNK))
    tab2 = _tc_detile(table.T)
    out5 = _emb_kernel(x3, tab2, pe2)
    return (out5.transpose(0, 2, 4, 1, 3)
            .reshape(_SEQ, _BATCH, _EMB))


# speedup vs baseline: 3.1072x; 3.1072x over previous
import functools

import jax
import jax.numpy as jnp
from jax import lax
from jax.experimental import pallas as pl
from jax.experimental.pallas import tpu as pltpu
from jax.experimental.pallas import tpu_sc as plsc

_VOCAB = 1000000
_EMB = 64
_SEQ = 200
_BATCH = 1024
_ROWS = _SEQ * _BATCH
_NW = 32
_CHUNK = 128
_NCHUNK = _ROWS // _CHUNK
_CPW = _NCHUNK // _NW
_CPP = _BATCH // _CHUNK
_LANES = 16
_VPR = _EMB // _LANES
_NBUF = 5
_KK = _CPW // _NBUF


def _pos_enc() -> jax.Array:
    i = jnp.arange(_SEQ, dtype=jnp.float32)[:, None]
    j = jnp.arange(_EMB)
    even = (j % 2 == 0)
    exponent = jnp.where(even, j, j - 1).astype(jnp.float32) / float(_EMB)
    angle = i / (10000.0 ** exponent)
    return jnp.where(even[None, :], jnp.sin(angle), jnp.cos(angle))


_MESH = plsc.VectorSubcoreMesh(core_axis_name="c", subcore_axis_name="s")


@functools.partial(
    pl.kernel,
    mesh=_MESH,
    out_type=jax.ShapeDtypeStruct((_SEQ, _BATCH, _EMB), jnp.float32),
    scratch_types=[
        pltpu.VMEM((_CPW, _CHUNK), jnp.int32),
        pltpu.VMEM((_NBUF, _CHUNK, _EMB), jnp.float32),
        pltpu.VMEM((_SEQ, _EMB), jnp.float32),
    ]
    + [pltpu.SemaphoreType.DMA] * (2 * _NBUF),
    compiler_params=pltpu.CompilerParams(use_tc_tiling_on_sc=False),
)
def _emb_kernel(x_hbm, table_hbm, pe_hbm, out_hbm, idx_v, rows_v, pe_v, *sems):
    gsem = sems[:_NBUF]
    osem = sems[_NBUF:]
    wid = lax.axis_index("s") * 2 + lax.axis_index("c")
    c0 = wid * _CPW
    pltpu.sync_copy(pe_hbm, pe_v)
    pltpu.sync_copy(x_hbm.at[wid], idx_v)

    def _gather(k, b):
        return pltpu.async_copy(
            table_hbm.at[idx_v.at[k]], rows_v.at[b], gsem[b])

    def _gather_wait(k, b):
        pltpu.make_async_copy(
            table_hbm.at[idx_v.at[k]], rows_v.at[b], gsem[b]).wait()

    def _out(k, b):
        c = c0 + k
        return pltpu.async_copy(
            rows_v.at[b],
            out_hbm.at[c // _CPP, pl.ds((c % _CPP) * _CHUNK, _CHUNK)],
            osem[b])

    def _out_wait(k, b):
        c = c0 + k
        pltpu.make_async_copy(
            rows_v.at[b],
            out_hbm.at[c // _CPP, pl.ds((c % _CPP) * _CHUNK, _CHUNK)],
            osem[b]).wait()

    _gather(0, 0)
    _gather(1, 1)

    def outer(kk, carry):
        for b in range(_NBUF):
            k = kk * _NBUF + b
            _gather_wait(k, b)
            s = (c0 + k) // _CPP
            pe_regs = [pe_v[s, pl.ds(j * _LANES, _LANES)]
                       for j in range(_VPR)]

            @plsc.parallel_loop(0, _CHUNK, 1, unroll=4)
            def add_body(r):
                for j in range(_VPR):
                    sl = pl.ds(j * _LANES, _LANES)
                    rows_v[b, r, sl] = rows_v[b, r, sl] + pe_regs[j]

            _out(k, b)
            b2 = (b + 2) % _NBUF
            if b < 3:
                @pl.when(kk >= 1)
                def _():
                    _out_wait(k - 3, b2)
            else:
                _out_wait(k - 3, b2)
            if b < 3:
                _gather(k + 2, b2)
            else:
                @pl.when(kk <= _KK - 2)
                def _():
                    _gather(k + 2, b2)
        return carry

    lax.fori_loop(0, _KK, outer, 0)
    for b, k in ((2, _CPW - 3), (3, _CPW - 2), (4, _CPW - 1)):
        _out_wait(k, b)


def kernel(x, table):
    pe = _pos_enc()
    return _emb_kernel(
        x.reshape(_NW, _CPW, _CHUNK).astype(jnp.int32), table, pe)
